# baseline (speedup 1.0000x reference)
import jax
import jax.numpy as jnp
from jax import lax
from jax.experimental import pallas as pl
from jax.experimental.pallas import tpu as pltpu, tpu_sc as plsc

_NC, _NS = 2, 16
_NW = _NC * _NS
_B = 16384
_DIM = 64
_PAD = 128
_OUTW = 3 * _DIM
_BPW = _B // _NW
_CHUNK = 64
_NR = _BPW // _CHUNK
_LANES = 16


def _compact_round(hbuf, rbuf, tbuf, obuf):
    for row in range(_CHUNK):
        out_base = row * _OUTW
        for t, buf in enumerate((hbuf, rbuf, tbuf)):
            for g in range(_DIM // _LANES):
                v = buf[row, pl.ds(g * _LANES, _LANES)]
                obuf[pl.ds(out_base + t * _DIM + g * _LANES, _LANES)] = v


def _gather_body(ent_hbm, rel_hbm, h_hbm, r_hbm, t_hbm, out_hbm,
                 hidx, ridx, tidx, hbuf, rbuf, tbuf, obuf, sem):
    wid = lax.axis_index("s") * _NC + lax.axis_index("c")
    base = wid * _BPW
    pltpu.sync_copy(h_hbm.at[pl.ds(base, _BPW)], hidx)
    pltpu.sync_copy(r_hbm.at[pl.ds(base, _BPW)], ridx)
    pltpu.sync_copy(t_hbm.at[pl.ds(base, _BPW)], tidx)

    def round_body(j):
        s = pl.ds(j * _CHUNK, _CHUNK)
        ch = pltpu.async_copy(ent_hbm.at[hidx.at[s]], hbuf, sem)
        cr = pltpu.async_copy(rel_hbm.at[ridx.at[s]], rbuf, sem)
        ct = pltpu.async_copy(ent_hbm.at[tidx.at[s]], tbuf, sem)
        ch.wait()
        cr.wait()
        ct.wait()
        _compact_round(hbuf, rbuf, tbuf, obuf)
        pltpu.sync_copy(
            obuf,
            out_hbm.at[pl.ds((base + j * _CHUNK) * _OUTW, _CHUNK * _OUTW)])

    lax.fori_loop(0, _NR, lambda j, _: (round_body(j), None)[1], None)


def kernel(entity_table, relation_table, h_idx, r_idx, t_idx, y):
    ent_pad = jnp.pad(entity_table, ((0, 0), (0, _PAD - _DIM)))
    rel_pad = jnp.pad(relation_table, ((0, 0), (0, _PAD - _DIM)))
    mesh = plsc.VectorSubcoreMesh(core_axis_name="c", subcore_axis_name="s")
    out = pl.kernel(
        _gather_body,
        out_type=jax.ShapeDtypeStruct((_B * _OUTW,), jnp.float32),
        mesh=mesh,
        compiler_params=pltpu.CompilerParams(use_tc_tiling_on_sc=True),
        scratch_types=[
            pltpu.VMEM((_BPW,), jnp.int32),
            pltpu.VMEM((_BPW,), jnp.int32),
            pltpu.VMEM((_BPW,), jnp.int32),
            pltpu.VMEM((_CHUNK, _PAD), jnp.float32),
            pltpu.VMEM((_CHUNK, _PAD), jnp.float32),
            pltpu.VMEM((_CHUNK, _PAD), jnp.float32),
            pltpu.VMEM((_CHUNK * _OUTW,), jnp.float32),
            pltpu.SemaphoreType.DMA,
        ],
    )(ent_pad, rel_pad,
      h_idx.astype(jnp.int32), r_idx.astype(jnp.int32), t_idx.astype(jnp.int32))
    return out
